# initial kernel scaffold (unmeasured)
import jax
import jax.numpy as jnp
from jax import lax
from jax.experimental import pallas as pl
from jax.experimental.pallas import tpu as pltpu


def kernel(
    x,
):
    def body(*refs):
        pass

    out_shape = jax.ShapeDtypeStruct(..., jnp.float32)
    return pl.pallas_call(body, out_shape=out_shape)(...)


# baseline (device time: 21694 ns/iter reference)
import jax
import jax.numpy as jnp
from jax import lax
from jax.experimental import pallas as pl
from jax.experimental.pallas import tpu as pltpu

N_DEV = 32
N_ROUNDS = 5


def kernel(x):
    m_per, n = x.shape

    def body(x_ref, out_ref, recv_buf, send_sems, recv_sems):
        my_pos = lax.axis_index("i")

        xv = x_ref[:, :]
        vmax = jnp.max(xv, axis=0)
        rows = lax.broadcasted_iota(jnp.int32, (m_per, n), 0)
        lidx = jnp.min(
            jnp.where(xv == vmax[None, :], rows, jnp.int32(m_per)), axis=0
        )
        out_ref[0, :] = vmax
        out_ref[1, :] = (my_pos * m_per + lidx).astype(jnp.float32)

        barrier_sem = pltpu.get_barrier_semaphore()
        for k in range(N_ROUNDS):
            pl.semaphore_signal(
                barrier_sem,
                inc=1,
                device_id=(my_pos ^ (1 << k),),
                device_id_type=pl.DeviceIdType.MESH,
            )
        pl.semaphore_wait(barrier_sem, N_ROUNDS)

        for k in range(N_ROUNDS):
            partner = my_pos ^ (1 << k)
            rdma = pltpu.make_async_remote_copy(
                src_ref=out_ref,
                dst_ref=recv_buf.at[k],
                send_sem=send_sems.at[k],
                recv_sem=recv_sems.at[k],
                device_id=(partner,),
                device_id_type=pl.DeviceIdType.MESH,
            )
            rdma.start()
            rdma.wait()

            v_t = recv_buf[k, 0, :]
            i_t = recv_buf[k, 1, :]
            v_m = out_ref[0, :]
            i_m = out_ref[1, :]
            take = (v_t > v_m) | ((v_t == v_m) & (i_t < i_m))
            out_ref[0, :] = jnp.where(take, v_t, v_m)
            out_ref[1, :] = jnp.where(take, i_t, i_m)

    return pl.pallas_call(
        body,
        out_shape=jax.ShapeDtypeStruct((2, n), jnp.float32),
        in_specs=[pl.BlockSpec(memory_space=pltpu.VMEM)],
        out_specs=pl.BlockSpec(memory_space=pltpu.VMEM),
        scratch_shapes=[
            pltpu.VMEM((N_ROUNDS, 2, n), jnp.float32),
            pltpu.SemaphoreType.DMA((N_ROUNDS,)),
            pltpu.SemaphoreType.DMA((N_ROUNDS,)),
        ],
        compiler_params=pltpu.CompilerParams(collective_id=0),
    )(x)


# device time: 18821 ns/iter; 1.1526x vs baseline; 1.1526x over previous
import jax
import jax.numpy as jnp
from jax import lax
from jax.experimental import pallas as pl
from jax.experimental.pallas import tpu as pltpu

N_DEV = 32
BIG = 1e9


def kernel(x):
    m_per, n = x.shape

    def body(x_ref, out_ref, local_ref, recv_buf, send_sems, recv_sems):
        my_pos = lax.axis_index("i")

        xv = x_ref[:, :]
        vmax_l = jnp.max(xv, axis=0)
        rows = lax.broadcasted_iota(jnp.int32, (m_per, n), 0)
        lidx = jnp.min(
            jnp.where(xv == vmax_l[None, :], rows, jnp.int32(m_per)), axis=0
        )
        gidx_l = (my_pos * m_per + lidx).astype(jnp.float32)
        local_ref[0, :] = vmax_l
        local_ref[1, :] = gidx_l

        barrier_sem = pltpu.get_barrier_semaphore()
        for j in range(N_DEV - 1):
            t = (my_pos + 1 + j) % N_DEV
            pl.semaphore_signal(
                barrier_sem, inc=1,
                device_id=(t,), device_id_type=pl.DeviceIdType.MESH,
            )
        pl.semaphore_wait(barrier_sem, N_DEV - 1)

        sends = []
        for j in range(N_DEV - 1):
            t = (my_pos + 1 + j) % N_DEV
            rdma = pltpu.make_async_remote_copy(
                src_ref=local_ref,
                dst_ref=recv_buf.at[my_pos],
                send_sem=send_sems.at[j],
                recv_sem=recv_sems.at[my_pos],
                device_id=(t,),
                device_id_type=pl.DeviceIdType.MESH,
            )
            rdma.start()
            sends.append(rdma)

        for j in range(N_DEV - 1):
            s = (my_pos + 1 + j) % N_DEV
            recv = pltpu.make_async_remote_copy(
                src_ref=local_ref,
                dst_ref=recv_buf.at[s],
                send_sem=send_sems.at[j],
                recv_sem=recv_sems.at[s],
                device_id=(s,),
                device_id_type=pl.DeviceIdType.MESH,
            )
            recv.wait_recv()
        for rdma in sends:
            rdma.wait_send()

        v = recv_buf[:, 0, :]
        i = recv_buf[:, 1, :]
        slot = lax.broadcasted_iota(jnp.int32, (N_DEV, n), 0)
        mine = slot == my_pos
        v = jnp.where(mine, jnp.float32(-jnp.inf), v)
        i = jnp.where(mine, BIG, i)
        vmax = jnp.maximum(jnp.max(v, axis=0), vmax_l)
        cand_r = jnp.min(jnp.where(v == vmax[None, :], i, BIG), axis=0)
        cand_l = jnp.where(vmax_l == vmax, gidx_l, BIG)
        out_ref[0, :] = vmax
        out_ref[1, :] = jnp.minimum(cand_r, cand_l)

    return pl.pallas_call(
        body,
        out_shape=jax.ShapeDtypeStruct((2, n), jnp.float32),
        in_specs=[pl.BlockSpec(memory_space=pltpu.VMEM)],
        out_specs=pl.BlockSpec(memory_space=pltpu.VMEM),
        scratch_shapes=[
            pltpu.VMEM((2, n), jnp.float32),
            pltpu.VMEM((N_DEV, 2, n), jnp.float32),
            pltpu.SemaphoreType.DMA((N_DEV - 1,)),
            pltpu.SemaphoreType.DMA((N_DEV,)),
        ],
        compiler_params=pltpu.CompilerParams(collective_id=0),
    )(x)
